# Initial kernel scaffold; baseline (speedup 1.0000x reference)
import jax
import jax.numpy as jnp
from jax.experimental import pallas as pl


def kernel(x, enc_W0, enc_b0, enc_W1, enc_b1, enc_W2, enc_b2, enc_W3, enc_b3, dec_W0, dec_b0, dec_W1, dec_b1, dec_W2, dec_b2, dec_W3, dec_b3, ln_g, ln_b, codebooks):
    raise NotImplementedError("write your pallas kernel here")


# speedup vs baseline: 1.2740x; 1.2740x over previous
import jax
import jax.numpy as jnp
from jax.experimental import pallas as pl
from jax.experimental.pallas import tpu as pltpu

_B = 16384
_IN = 768
_HID = 32
_K = 256
_L = 3
_BETA = 0.25
_TILE = 512


def _mm(a, b):
    return jnp.dot(a, b, preferred_element_type=jnp.float32)


def _fused(x_ref,
           eW0, eb0, eW1, eb1, eW2, eb2, eW3, eb3,
           dW0, db0, dW1, db1, dW2, db2, dW3, db3,
           ln_g, ln_b, cb_ref, cbT_ref,
           out_ref, i0_ref, i1_ref, i2_ref, q_ref, loss_ref):
    f32 = jnp.float32
    h = x_ref[...]
    h = jnp.maximum(_mm(h, eW0[...]) + eb0[...], 0.0)
    h = jnp.maximum(_mm(h, eW1[...]) + eb1[...], 0.0)
    h = jnp.maximum(_mm(h, eW2[...]) + eb2[...], 0.0)
    h = _mm(h, eW3[...]) + eb3[...]
    mu = jnp.mean(h, axis=-1, keepdims=True)
    var = jnp.mean((h - mu) ** 2, axis=-1, keepdims=True)
    encoded = (h - mu) / jnp.sqrt(var + 1e-5) * ln_g[...] + ln_b[...]

    iota_k = jax.lax.broadcasted_iota(jnp.int32, (_TILE, _K), 1)
    idx_refs = (i0_ref, i1_ref, i2_ref)
    resid = encoded
    qrep = jnp.zeros_like(encoded)
    loss_sum = jnp.zeros((), f32)
    for level in range(_L):
        cb = cb_ref[level]
        cbT = cbT_ref[level]
        x2 = jnp.sum(resid * resid, axis=1, keepdims=True)
        y2 = jnp.sum(cb * cb, axis=1)[None, :]
        d = x2 + y2 - 2.0 * _mm(resid, cbT)
        dmin = jnp.min(d, axis=1, keepdims=True)
        idx = jnp.min(jnp.where(d == dmin, iota_k, _K), axis=1)
        onehot = (iota_k == idx[:, None]).astype(f32)
        qv = jnp.dot(onehot, cb, precision=jax.lax.Precision.HIGHEST,
                     preferred_element_type=f32)
        loss_sum = loss_sum + jnp.sum((resid - qv) ** 2)
        qrep = qrep + qv
        idx_refs[level][...] = idx[:, None]
        resid = resid - qv
    loss_ref[...] = loss_sum.reshape(1, 1, 1)

    q_st = encoded + (qrep - encoded)
    q_ref[...] = q_st
    h = q_st
    h = jnp.maximum(_mm(h, dW0[...]) + db0[...], 0.0)
    h = jnp.maximum(_mm(h, dW1[...]) + db1[...], 0.0)
    h = jnp.maximum(_mm(h, dW2[...]) + db2[...], 0.0)
    out_ref[...] = _mm(h, dW3[...]) + db3[...]


def kernel(x, enc_W0, enc_b0, enc_W1, enc_b1, enc_W2, enc_b2, enc_W3, enc_b3,
           dec_W0, dec_b0, dec_W1, dec_b1, dec_W2, dec_b2, dec_W3, dec_b3,
           ln_g, ln_b, codebooks):
    grid = _B // _TILE
    cbT = jnp.swapaxes(codebooks, 1, 2)

    def row_spec(cols):
        return pl.BlockSpec((_TILE, cols), lambda i: (i, 0))

    def full_spec(arr):
        nd = arr.ndim
        return pl.BlockSpec(arr.shape, lambda i, _n=nd: (0,) * _n)

    biases = [b.reshape(1, -1) for b in
              (enc_b0, enc_b1, enc_b2, enc_b3, dec_b0, dec_b1, dec_b2, dec_b3)]
    ln_g2 = ln_g.reshape(1, -1)
    ln_b2 = ln_b.reshape(1, -1)

    operands = (x,
                enc_W0, biases[0], enc_W1, biases[1],
                enc_W2, biases[2], enc_W3, biases[3],
                dec_W0, biases[4], dec_W1, biases[5],
                dec_W2, biases[6], dec_W3, biases[7],
                ln_g2, ln_b2, codebooks, cbT)
    in_specs = [row_spec(_IN)] + [full_spec(a) for a in operands[1:]]

    out_shape = (
        jax.ShapeDtypeStruct((_B, _IN), jnp.float32),
        jax.ShapeDtypeStruct((_B, 1), jnp.int32),
        jax.ShapeDtypeStruct((_B, 1), jnp.int32),
        jax.ShapeDtypeStruct((_B, 1), jnp.int32),
        jax.ShapeDtypeStruct((_B, _HID), jnp.float32),
        jax.ShapeDtypeStruct((grid, 1, 1), jnp.float32),
    )
    out_specs = (
        row_spec(_IN),
        row_spec(1), row_spec(1), row_spec(1),
        row_spec(_HID),
        pl.BlockSpec((1, 1, 1), lambda i: (i, 0, 0)),
    )

    h, i0, i1, i2, q_st, loss_parts = pl.pallas_call(
        _fused,
        grid=(grid,),
        in_specs=in_specs,
        out_specs=out_specs,
        out_shape=out_shape,
        compiler_params=pltpu.CompilerParams(
            dimension_semantics=("parallel",),
        ),
    )(*operands)

    code_indices = jnp.concatenate([i0, i1, i2], axis=1)
    loss = (_BETA / (_B * _HID)) * jnp.sum(loss_parts)
    return (h, code_indices, q_st, loss)


# speedup vs baseline: 1.4368x; 1.1277x over previous
import jax
import jax.numpy as jnp
from jax.experimental import pallas as pl
from jax.experimental.pallas import tpu as pltpu

_B = 16384
_IN = 768
_HID = 32
_K = 256
_L = 3
_BETA = 0.25
_TILE = 1024


def _mm(a, b):
    return jnp.dot(a, b, preferred_element_type=jnp.float32)


def _fused(x_ref,
           eW0, eb0, eW1, eb1, eW2, eb2, eW3, eb3,
           dW0, db0, dW1, db1, dW2, db2, dW3, db3,
           ln_g, ln_b, cb_ref, cbT_ref,
           out_ref, i0_ref, i1_ref, i2_ref, q_ref, loss_ref):
    f32 = jnp.float32
    h = x_ref[...]
    h = jnp.maximum(_mm(h, eW0[...]) + eb0[...], 0.0)
    h = jnp.maximum(_mm(h, eW1[...]) + eb1[...], 0.0)
    h = jnp.maximum(_mm(h, eW2[...]) + eb2[...], 0.0)
    h = _mm(h, eW3[...]) + eb3[...]
    mu = jnp.mean(h, axis=-1, keepdims=True)
    var = jnp.mean((h - mu) ** 2, axis=-1, keepdims=True)
    encoded = (h - mu) / jnp.sqrt(var + 1e-5) * ln_g[...] + ln_b[...]

    iota_k = jax.lax.broadcasted_iota(jnp.int32, (_TILE, _K), 1)
    idx_refs = (i0_ref, i1_ref, i2_ref)
    resid = encoded
    qrep = jnp.zeros_like(encoded)
    loss_sum = jnp.zeros((), f32)
    for level in range(_L):
        cb = cb_ref[level]
        x2 = jnp.sum(resid * resid, axis=1, keepdims=True)
        y2 = jnp.sum(cb * cb, axis=1)[None, :]
        d = x2 + y2 - 2.0 * _mm(resid, cbT_ref[level])
        dmin = jnp.min(d, axis=1, keepdims=True)
        idx = jnp.min(jnp.where(d == dmin, iota_k, _K), axis=1)
        onehot = (iota_k == idx[:, None]).astype(f32)
        qv = jnp.dot(onehot, cb, precision=jax.lax.Precision.HIGHEST,
                     preferred_element_type=f32)
        loss_sum = loss_sum + jnp.sum((resid - qv) ** 2)
        qrep = qrep + qv
        idx_refs[level][...] = idx[:, None]
        resid = resid - qv
    loss_ref[...] = loss_sum.reshape(1, 1, 1)

    q_st = encoded + (qrep - encoded)
    q_ref[...] = q_st
    h = q_st
    h = jnp.maximum(_mm(h, dW0[...]) + db0[...], 0.0)
    h = jnp.maximum(_mm(h, dW1[...]) + db1[...], 0.0)
    h = jnp.maximum(_mm(h, dW2[...]) + db2[...], 0.0)
    out_ref[...] = _mm(h, dW3[...]) + db3[...]


def kernel(x, enc_W0, enc_b0, enc_W1, enc_b1, enc_W2, enc_b2, enc_W3, enc_b3,
           dec_W0, dec_b0, dec_W1, dec_b1, dec_W2, dec_b2, dec_W3, dec_b3,
           ln_g, ln_b, codebooks):
    grid = _B // _TILE
    cbT = jnp.swapaxes(codebooks, 1, 2)

    def row_spec(cols):
        return pl.BlockSpec((_TILE, cols), lambda i: (i, 0))

    def full_spec(arr):
        nd = arr.ndim
        return pl.BlockSpec(arr.shape, lambda i, _n=nd: (0,) * _n)

    biases = [b.reshape(1, -1) for b in
              (enc_b0, enc_b1, enc_b2, enc_b3, dec_b0, dec_b1, dec_b2, dec_b3)]
    ln_g2 = ln_g.reshape(1, -1)
    ln_b2 = ln_b.reshape(1, -1)

    operands = (x,
                enc_W0, biases[0], enc_W1, biases[1],
                enc_W2, biases[2], enc_W3, biases[3],
                dec_W0, biases[4], dec_W1, biases[5],
                dec_W2, biases[6], dec_W3, biases[7],
                ln_g2, ln_b2, codebooks, cbT)
    in_specs = [row_spec(_IN)] + [full_spec(a) for a in operands[1:]]

    out_shape = (
        jax.ShapeDtypeStruct((_B, _IN), jnp.float32),
        jax.ShapeDtypeStruct((_B, 1), jnp.int32),
        jax.ShapeDtypeStruct((_B, 1), jnp.int32),
        jax.ShapeDtypeStruct((_B, 1), jnp.int32),
        jax.ShapeDtypeStruct((_B, _HID), jnp.float32),
        jax.ShapeDtypeStruct((grid, 1, 1), jnp.float32),
    )
    out_specs = (
        row_spec(_IN),
        row_spec(1), row_spec(1), row_spec(1),
        row_spec(_HID),
        pl.BlockSpec((1, 1, 1), lambda i: (i, 0, 0)),
    )

    h, i0, i1, i2, q_st, loss_parts = pl.pallas_call(
        _fused,
        grid=(grid,),
        in_specs=in_specs,
        out_specs=out_specs,
        out_shape=out_shape,
        compiler_params=pltpu.CompilerParams(
            dimension_semantics=("parallel",),
        ),
    )(*operands)

    code_indices = jnp.concatenate([i0, i1, i2], axis=1)
    loss = (_BETA / (_B * _HID)) * jnp.sum(loss_parts)
    return (h, code_indices, q_st, loss)
